# baseline (speedup 1.0000x reference)
import jax
import jax.numpy as jnp
import numpy as np
from jax.experimental import pallas as pl
from jax.experimental.pallas import tpu as pltpu

B = 64
IN_FEATS = 64
NUM_NODES = 16
D_NODE = 4
HID = 128
H = W = 224
WPAD = 256
HPAD = 232


def _fused_kernel(x_ref, w_ref, b_ref, d_ref, sel_ref, atd_ref, abu_ref,
                  wtd_ref, btd_ref, wbu_ref, bbu_ref, wg2_ref, bg2_ref,
                  wfc_ref, bfc_ref, out_ref):
    def mm(a, b):
        return jax.lax.dot_general(a, b, (((1,), (0,)), ((), ())),
                                   preferred_element_type=jnp.float32)

    def body(c, acc):
        xs_blk = x_ref[0, :, pl.ds(c * 8, 16), :]
        p = jnp.concatenate(
            [xs_blk[:, :, kw:kw + WPAD].reshape(48, WPAD) for kw in range(3)],
            axis=0)
        r = mm(w_ref[:], p)
        r = jnp.maximum(r + b_ref[:], 0.0)
        return acc + jnp.sum(r.reshape(8, IN_FEATS, WPAD), axis=0)

    acc = jax.lax.fori_loop(0, H // 8, body,
                            jnp.zeros((IN_FEATS, WPAD), jnp.float32))
    mask = (jax.lax.broadcasted_iota(jnp.int32, (1, WPAD), 1) < W)
    acc = jnp.where(mask, acc, 0.0)
    pooled = jnp.sum(acc, axis=1, keepdims=True) * (1.0 / (H * W))

    hs = mm(sel_ref[:], pooled * d_ref[:])
    tdn = mm(atd_ref[:], hs)
    bun = mm(abu_ref[:], hs)
    td = jnp.maximum(mm(tdn, wtd_ref[:]) + btd_ref[:], 0.0)
    bu = jnp.maximum(mm(bun, wbu_ref[:]) + bbu_ref[:], 0.0)
    z = jnp.concatenate([td, bu], axis=1)
    z2 = jnp.maximum(mm(mm(atd_ref[:], z), wg2_ref[:]) + bg2_ref[:], 0.0)
    g = jnp.sum(z2, axis=0, keepdims=True) * (1.0 / NUM_NODES)
    out_ref[0] = mm(g, wfc_ref[:]) + bfc_ref[:]


def kernel(x, W_conv, b_conv, W_td, b_td, W_bu, b_bu, W_g2, b_g2, W_fc, b_fc):
    xp = jnp.pad(x.astype(jnp.bfloat16),
                 ((0, 0), (0, 0), (1, HPAD - H - 1), (1, WPAD + 2 - W - 1)))
    shift = ((np.arange(16)[None, :, None] - np.arange(8)[:, None, None])
             == np.arange(3)[None, None, :]).astype(np.float32)
    w2 = jnp.einsum('oihw,drh->dowir', W_conv,
                    jnp.asarray(shift)).reshape(8 * IN_FEATS, 144)
    w2 = w2.astype(jnp.bfloat16)
    bc = jnp.tile(b_conv, 8).reshape(8 * IN_FEATS, 1)

    f = np.arange(IN_FEATS)
    dmat = jnp.asarray((f[:, None] % D_NODE) == np.arange(D_NODE)[None, :],
                       jnp.float32)
    sel = jnp.asarray((f[None, :] // D_NODE) == np.arange(NUM_NODES)[:, None],
                      jnp.float32)

    idx = np.arange(NUM_NODES)
    atd = jnp.asarray(np.where(idx[None, :] > idx[:, None],
                               1.0 / np.maximum(NUM_NODES - 1 - idx, 1)[:, None],
                               0.0), jnp.float32)
    abu = jnp.asarray(np.where(idx[None, :] < idx[:, None],
                               1.0 / np.maximum(idx, 1)[:, None],
                               0.0), jnp.float32)

    num_classes = W_fc.shape[1]
    full = lambda shape: pl.BlockSpec(shape, lambda i: tuple(0 for _ in shape))
    out = pl.pallas_call(
        _fused_kernel,
        grid=(B,),
        in_specs=[
            pl.BlockSpec((1, 3, HPAD, WPAD + 2), lambda i: (i, 0, 0, 0)),
            full((8 * IN_FEATS, 144)),
            full((8 * IN_FEATS, 1)),
            full((IN_FEATS, D_NODE)),
            full((NUM_NODES, IN_FEATS)),
            full((NUM_NODES, NUM_NODES)),
            full((NUM_NODES, NUM_NODES)),
            full((D_NODE, HID)),
            full((1, HID)),
            full((D_NODE, HID)),
            full((1, HID)),
            full((2 * HID, HID)),
            full((1, HID)),
            full((HID, num_classes)),
            full((1, num_classes)),
        ],
        out_specs=pl.BlockSpec((1, 1, num_classes), lambda i: (i, 0, 0)),
        out_shape=jax.ShapeDtypeStruct((B, 1, num_classes), jnp.float32),
        compiler_params=pltpu.CompilerParams(
            dimension_semantics=("parallel",)),
    )(xp, w2, bc, dmat, sel, atd, abu, W_td, b_td.reshape(1, HID), W_bu,
      b_bu.reshape(1, HID), W_g2, b_g2.reshape(1, HID), W_fc,
      b_fc.reshape(1, num_classes))
    return out.reshape(B, num_classes)


# speedup vs baseline: 2.5168x; 2.5168x over previous
import jax
import jax.numpy as jnp
import numpy as np
from jax.experimental import pallas as pl
from jax.experimental.pallas import tpu as pltpu

B = 64
IN_FEATS = 64
NUM_NODES = 16
D_NODE = 4
HID = 128
H = W = 224
WPAD = 256
HPAD = 232


def _fused_kernel(x_ref, w_ref, b_ref, d_ref, sel_ref, atd_ref, abu_ref,
                  wtd_ref, btd_ref, wbu_ref, bbu_ref, wg2_ref, bg2_ref,
                  wfc_ref, bfc_ref, out_ref):
    def mm(a, b):
        return jax.lax.dot_general(a, b, (((1,), (0,)), ((), ())),
                                   preferred_element_type=jnp.float32)

    accs = [jnp.zeros((IN_FEATS, WPAD), jnp.float32) for _ in range(4)]
    for c in range(H // 8):
        xs_blk = x_ref[0, :, c * 8:c * 8 + 16, :]
        p = jnp.concatenate(
            [xs_blk[:, :, kw:kw + WPAD].reshape(48, WPAD) for kw in range(3)],
            axis=0)
        r = mm(w_ref[:], p)
        r = jnp.maximum(r + b_ref[:], 0.0)
        accs[c % 4] = accs[c % 4] + jnp.sum(r.reshape(8, IN_FEATS, WPAD),
                                            axis=0)
    acc = (accs[0] + accs[1]) + (accs[2] + accs[3])
    mask = (jax.lax.broadcasted_iota(jnp.int32, (1, WPAD), 1) < W)
    acc = jnp.where(mask, acc, 0.0)
    pooled = jnp.sum(acc, axis=1, keepdims=True) * (1.0 / (H * W))

    hs = mm(sel_ref[:], pooled * d_ref[:])
    tdn = mm(atd_ref[:], hs)
    bun = mm(abu_ref[:], hs)
    td = jnp.maximum(mm(tdn, wtd_ref[:]) + btd_ref[:], 0.0)
    bu = jnp.maximum(mm(bun, wbu_ref[:]) + bbu_ref[:], 0.0)
    z = jnp.concatenate([td, bu], axis=1)
    z2 = jnp.maximum(mm(mm(atd_ref[:], z), wg2_ref[:]) + bg2_ref[:], 0.0)
    g = jnp.sum(z2, axis=0, keepdims=True) * (1.0 / NUM_NODES)
    out_ref[0] = mm(g, wfc_ref[:]) + bfc_ref[:]


def kernel(x, W_conv, b_conv, W_td, b_td, W_bu, b_bu, W_g2, b_g2, W_fc, b_fc):
    xp = jnp.pad(x.astype(jnp.bfloat16),
                 ((0, 0), (0, 0), (1, HPAD - H - 1), (1, WPAD + 2 - W - 1)))
    shift = ((np.arange(16)[None, :, None] - np.arange(8)[:, None, None])
             == np.arange(3)[None, None, :]).astype(np.float32)
    w2 = jnp.einsum('oihw,drh->dowir', W_conv,
                    jnp.asarray(shift)).reshape(8 * IN_FEATS, 144)
    w2 = w2.astype(jnp.bfloat16)
    bc = jnp.tile(b_conv, 8).reshape(8 * IN_FEATS, 1)

    f = np.arange(IN_FEATS)
    dmat = jnp.asarray((f[:, None] % D_NODE) == np.arange(D_NODE)[None, :],
                       jnp.float32)
    sel = jnp.asarray((f[None, :] // D_NODE) == np.arange(NUM_NODES)[:, None],
                      jnp.float32)

    idx = np.arange(NUM_NODES)
    atd = jnp.asarray(np.where(idx[None, :] > idx[:, None],
                               1.0 / np.maximum(NUM_NODES - 1 - idx, 1)[:, None],
                               0.0), jnp.float32)
    abu = jnp.asarray(np.where(idx[None, :] < idx[:, None],
                               1.0 / np.maximum(idx, 1)[:, None],
                               0.0), jnp.float32)

    num_classes = W_fc.shape[1]
    full = lambda shape: pl.BlockSpec(shape, lambda i: tuple(0 for _ in shape))
    out = pl.pallas_call(
        _fused_kernel,
        grid=(B,),
        in_specs=[
            pl.BlockSpec((1, 3, HPAD, WPAD + 2), lambda i: (i, 0, 0, 0)),
            full((8 * IN_FEATS, 144)),
            full((8 * IN_FEATS, 1)),
            full((IN_FEATS, D_NODE)),
            full((NUM_NODES, IN_FEATS)),
            full((NUM_NODES, NUM_NODES)),
            full((NUM_NODES, NUM_NODES)),
            full((D_NODE, HID)),
            full((1, HID)),
            full((D_NODE, HID)),
            full((1, HID)),
            full((2 * HID, HID)),
            full((1, HID)),
            full((HID, num_classes)),
            full((1, num_classes)),
        ],
        out_specs=pl.BlockSpec((1, 1, num_classes), lambda i: (i, 0, 0)),
        out_shape=jax.ShapeDtypeStruct((B, 1, num_classes), jnp.float32),
        compiler_params=pltpu.CompilerParams(
            dimension_semantics=("parallel",)),
    )(xp, w2, bc, dmat, sel, atd, abu, W_td, b_td.reshape(1, HID), W_bu,
      b_bu.reshape(1, HID), W_g2, b_g2.reshape(1, HID), W_fc,
      b_fc.reshape(1, num_classes))
    return out.reshape(B, num_classes)


# speedup vs baseline: 2.7140x; 1.0783x over previous
import jax
import jax.numpy as jnp
import numpy as np
from jax.experimental import pallas as pl
from jax.experimental.pallas import tpu as pltpu

B = 64
IN_FEATS = 64
NUM_NODES = 16
D_NODE = 4
HID = 128
H = W = 224
WPAD = 256
HPAD = 232


def _fused_kernel(x_ref, w_ref, b_ref, d_ref, sel_ref, atd_ref, abu_ref,
                  wtd_ref, btd_ref, wbu_ref, bbu_ref, wg2_ref, bg2_ref,
                  wfc_ref, bfc_ref, out_ref):
    def mm(a, b):
        return jax.lax.dot_general(a, b, (((1,), (0,)), ((), ())),
                                   preferred_element_type=jnp.float32)

    accs = [jnp.zeros((IN_FEATS, WPAD), jnp.float32) for _ in range(4)]
    ones_row = jnp.ones((1, WPAD), jnp.bfloat16)
    for c in range(H // 8):
        xs_blk = x_ref[0, :, c * 8:c * 8 + 16, :]
        p = jnp.concatenate(
            [xs_blk[:, :, kw:kw + WPAD].reshape(48, WPAD) for kw in range(3)]
            + [ones_row],
            axis=0)
        r = mm(w_ref[:], p)
        r = jnp.maximum(r, 0.0)
        accs[c % 4] = accs[c % 4] + jnp.sum(r.reshape(8, IN_FEATS, WPAD),
                                            axis=0)
    acc = (accs[0] + accs[1]) + (accs[2] + accs[3])
    mask = (jax.lax.broadcasted_iota(jnp.int32, (1, WPAD), 1) < W)
    acc = jnp.where(mask, acc, 0.0)
    pooled = jnp.sum(acc, axis=1, keepdims=True) * (1.0 / (H * W))

    hs = mm(sel_ref[:], pooled * d_ref[:])
    tdn = mm(atd_ref[:], hs)
    bun = mm(abu_ref[:], hs)
    td = jnp.maximum(mm(tdn, wtd_ref[:]) + btd_ref[:], 0.0)
    bu = jnp.maximum(mm(bun, wbu_ref[:]) + bbu_ref[:], 0.0)
    z = jnp.concatenate([td, bu], axis=1)
    z2 = jnp.maximum(mm(mm(atd_ref[:], z), wg2_ref[:]) + bg2_ref[:], 0.0)
    g = jnp.sum(z2, axis=0, keepdims=True) * (1.0 / NUM_NODES)
    out_ref[0] = mm(g, wfc_ref[:]) + bfc_ref[:]


def kernel(x, W_conv, b_conv, W_td, b_td, W_bu, b_bu, W_g2, b_g2, W_fc, b_fc):
    xp = jnp.pad(x.astype(jnp.bfloat16),
                 ((0, 0), (0, 0), (1, HPAD - H - 1), (1, WPAD + 2 - W - 1)))
    shift = ((np.arange(16)[None, :, None] - np.arange(8)[:, None, None])
             == np.arange(3)[None, None, :]).astype(np.float32)
    w2 = jnp.einsum('oihw,drh->dowir', W_conv,
                    jnp.asarray(shift)).reshape(8 * IN_FEATS, 144)
    bc = jnp.tile(b_conv, 8).reshape(8 * IN_FEATS, 1)
    w2 = jnp.concatenate([w2, bc], axis=1).astype(jnp.bfloat16)

    f = np.arange(IN_FEATS)
    dmat = jnp.asarray((f[:, None] % D_NODE) == np.arange(D_NODE)[None, :],
                       jnp.float32)
    sel = jnp.asarray((f[None, :] // D_NODE) == np.arange(NUM_NODES)[:, None],
                      jnp.float32)

    idx = np.arange(NUM_NODES)
    atd = jnp.asarray(np.where(idx[None, :] > idx[:, None],
                               1.0 / np.maximum(NUM_NODES - 1 - idx, 1)[:, None],
                               0.0), jnp.float32)
    abu = jnp.asarray(np.where(idx[None, :] < idx[:, None],
                               1.0 / np.maximum(idx, 1)[:, None],
                               0.0), jnp.float32)

    num_classes = W_fc.shape[1]
    full = lambda shape: pl.BlockSpec(shape, lambda i: tuple(0 for _ in shape))
    out = pl.pallas_call(
        _fused_kernel,
        grid=(B,),
        in_specs=[
            pl.BlockSpec((1, 3, HPAD, WPAD + 2), lambda i: (i, 0, 0, 0)),
            full((8 * IN_FEATS, 145)),
            full((8 * IN_FEATS, 1)),
            full((IN_FEATS, D_NODE)),
            full((NUM_NODES, IN_FEATS)),
            full((NUM_NODES, NUM_NODES)),
            full((NUM_NODES, NUM_NODES)),
            full((D_NODE, HID)),
            full((1, HID)),
            full((D_NODE, HID)),
            full((1, HID)),
            full((2 * HID, HID)),
            full((1, HID)),
            full((HID, num_classes)),
            full((1, num_classes)),
        ],
        out_specs=pl.BlockSpec((1, 1, num_classes), lambda i: (i, 0, 0)),
        out_shape=jax.ShapeDtypeStruct((B, 1, num_classes), jnp.float32),
        compiler_params=pltpu.CompilerParams(
            dimension_semantics=("parallel",)),
    )(xp, w2, bc, dmat, sel, atd, abu, W_td, b_td.reshape(1, HID), W_bu,
      b_bu.reshape(1, HID), W_g2, b_g2.reshape(1, HID), W_fc,
      b_fc.reshape(1, num_classes))
    return out.reshape(B, num_classes)


# speedup vs baseline: 3.1482x; 1.1600x over previous
import jax
import jax.numpy as jnp
import numpy as np
from jax.experimental import pallas as pl
from jax.experimental.pallas import tpu as pltpu

B = 64
IN_FEATS = 64
NUM_NODES = 16
D_NODE = 4
HID = 128
H = W = 224
WPAD = 256
HPAD = 232
NB = NUM_NODES * B


def _conv_pool_kernel(x_ref, w_ref, out_ref):
    def mm(a, b):
        return jax.lax.dot_general(a, b, (((1,), (0,)), ((), ())),
                                   preferred_element_type=jnp.float32)

    accs = [jnp.zeros((IN_FEATS, WPAD), jnp.float32) for _ in range(4)]
    ones_row = jnp.ones((1, WPAD), jnp.bfloat16)
    for c in range(H // 8):
        xs_blk = x_ref[0, :, c * 8:c * 8 + 16, :]
        p = jnp.concatenate(
            [xs_blk[:, :, kw:kw + WPAD].reshape(48, WPAD) for kw in range(3)]
            + [ones_row],
            axis=0)
        r = mm(w_ref[:], p)
        r = jnp.maximum(r, 0.0)
        accs[c % 4] = accs[c % 4] + jnp.sum(r.reshape(8, IN_FEATS, WPAD),
                                            axis=0)
    acc = (accs[0] + accs[1]) + (accs[2] + accs[3])
    mask = (jax.lax.broadcasted_iota(jnp.int32, (1, WPAD), 1) < W)
    acc = jnp.where(mask, acc, 0.0)
    out_ref[0] = jnp.sum(acc, axis=1, keepdims=True) * (1.0 / (H * W))


def _gcn_head_kernel(h2_ref, atd_ref, abu_ref, mpool_ref, wtd_ref, btd_ref,
                     wbu_ref, bbu_ref, wg2_ref, bg2_ref, wfc_ref, bfc_ref,
                     out_ref):
    def mm(a, b):
        return jax.lax.dot_general(a, b, (((1,), (0,)), ((), ())),
                                   preferred_element_type=jnp.float32)

    h2 = h2_ref[:]
    td = jnp.maximum(mm(atd_ref[:], mm(h2, wtd_ref[:])) + btd_ref[:], 0.0)
    bu = jnp.maximum(mm(abu_ref[:], mm(h2, wbu_ref[:])) + bbu_ref[:], 0.0)
    z = jnp.concatenate([td, bu], axis=1)
    z2 = jnp.maximum(mm(atd_ref[:], mm(z, wg2_ref[:])) + bg2_ref[:], 0.0)
    out_ref[:] = mm(mm(mpool_ref[:], z2), wfc_ref[:]) + bfc_ref[:]


def kernel(x, W_conv, b_conv, W_td, b_td, W_bu, b_bu, W_g2, b_g2, W_fc, b_fc):
    xp = jnp.pad(x.astype(jnp.bfloat16),
                 ((0, 0), (0, 0), (1, HPAD - H - 1), (1, WPAD + 2 - W - 1)))
    shift = ((np.arange(16)[None, :, None] - np.arange(8)[:, None, None])
             == np.arange(3)[None, None, :]).astype(np.float32)
    w2 = jnp.einsum('oihw,drh->dowir', W_conv,
                    jnp.asarray(shift)).reshape(8 * IN_FEATS, 144)
    bc = jnp.tile(b_conv, 8).reshape(8 * IN_FEATS, 1)
    w2 = jnp.concatenate([w2, bc], axis=1).astype(jnp.bfloat16)

    full = lambda shape: pl.BlockSpec(shape, lambda i: tuple(0 for _ in shape))
    pooled = pl.pallas_call(
        _conv_pool_kernel,
        grid=(B,),
        in_specs=[
            pl.BlockSpec((1, 3, HPAD, WPAD + 2), lambda i: (i, 0, 0, 0)),
            full((8 * IN_FEATS, 145)),
        ],
        out_specs=pl.BlockSpec((1, IN_FEATS, 1), lambda i: (i, 0, 0)),
        out_shape=jax.ShapeDtypeStruct((B, IN_FEATS, 1), jnp.float32),
        compiler_params=pltpu.CompilerParams(
            dimension_semantics=("parallel",)),
    )(xp, w2)

    h2 = pooled.reshape(B, NUM_NODES, D_NODE).transpose(1, 0, 2).reshape(
        NB, D_NODE)

    idx = np.arange(NUM_NODES)
    atd = np.where(idx[None, :] > idx[:, None],
                   1.0 / np.maximum(NUM_NODES - 1 - idx, 1)[:, None], 0.0)
    abu = np.where(idx[None, :] < idx[:, None],
                   1.0 / np.maximum(idx, 1)[:, None], 0.0)
    eye = np.eye(B, dtype=np.float32)
    atd_big = jnp.asarray(np.kron(atd, eye), jnp.float32)
    abu_big = jnp.asarray(np.kron(abu, eye), jnp.float32)
    mpool = jnp.asarray(np.kron(np.full((1, NUM_NODES), 1.0 / NUM_NODES,
                                        np.float32), eye), jnp.float32)

    num_classes = W_fc.shape[1]
    out = pl.pallas_call(
        _gcn_head_kernel,
        out_shape=jax.ShapeDtypeStruct((B, num_classes), jnp.float32),
    )(h2, atd_big, abu_big, mpool, W_td, b_td.reshape(1, HID), W_bu,
      b_bu.reshape(1, HID), W_g2, b_g2.reshape(1, HID), W_fc,
      b_fc.reshape(1, num_classes))
    return out
